# baseline (speedup 1.0000x reference)
import jax
import jax.numpy as jnp
from jax import lax
from jax.experimental import pallas as pl
from jax.experimental.pallas import tpu as pltpu
from jax.experimental.pallas import tpu_sc as plsc

NUM_NTYPE = 4
TBL = 250000
EMBED = 64
B = 16384

NC, NS, L = 2, 16, 16
NW = NC * NS

PANW = 128
NPAN_T = (TBL + PANW - 1) // PANW
JTAIL = NPAN_T - 1
NPAN = NUM_NTYPE * NPAN_T
MYPAN = (NPAN + NW - 1) // NW
CAP = 20
MMAX = 1024
SCAP = 64
RING = 6
LAG = 3


def _body(tids_hbm, xids_hbm, e0, e1, e2, e3, out_hbm,
          tv_all, xv_all, cl_lp, cl_pb, spc, mcnt, mslots,
          pbufs, rowbufs, dbuf, ptail, sems, semos, semsp):
    embs = (e0, e1, e2, e3)
    wid = lax.axis_index("s") * NC + lax.axis_index("c")

    pltpu.sync_copy(tids_hbm, tv_all)
    pltpu.sync_copy(xids_hbm, xv_all)

    zeros = jnp.zeros((L,), jnp.int32)
    for z in range((MYPAN + 2 * L) // L):
        mcnt[pl.ds(z * L, L)] = zeros

    iota = lax.iota(jnp.int32, L)

    def scan(i, offs):
        off, soff = offs
        tvv = tv_all[pl.ds(i * L, L)]
        xvv = xv_all[pl.ds(i * L, L)]
        j = xvv >> 7
        p = tvv * jnp.int32(NPAN_T) + j
        mine = (p & jnp.int32(NW - 1)) == wid
        tail = j == jnp.int32(JTAIL)
        minen = mine & (~tail)
        mines = mine & tail
        bvec = i * L + iota
        pb = (bvec << 7) | (xvv & jnp.int32(127))
        spb = (bvec << 9) | (tvv << 7) | (xvv & jnp.int32(127))

        incn = minen.astype(jnp.int32)
        cumn = plsc.cumsum(incn)
        destn = jnp.where(minen, off + cumn - incn,
                          jnp.int32(MMAX + L) + iota)
        plsc.store_scatter(cl_lp, [destn], p >> 5)
        plsc.store_scatter(cl_pb, [destn], pb)

        incs = mines.astype(jnp.int32)
        cums = plsc.cumsum(incs)
        dests = jnp.where(mines, soff + cums - incs,
                          jnp.int32(SCAP + L) + iota)
        plsc.store_scatter(spc, [dests], spb)
        return (off + cumn[L - 1], soff + cums[L - 1])
    m, sm = lax.fori_loop(0, B // L, scan, (jnp.int32(0), jnp.int32(0)))

    def fill(g, _):
        lpv = cl_lp[pl.ds(g * L, L)]
        pbv = cl_pb[pl.ds(g * L, L)]
        for kk in range(L):
            idx = g * L + kk

            @pl.when(idx < m)
            def _(kk=kk, lpv=lpv, pbv=pbv):
                lp = lpv[kk]
                pb = pbv[kk]
                cnt = mcnt[pl.ds(lp, L)][0]
                plsc.store_scatter(
                    mslots, [jnp.full((L,), lp * CAP + cnt, jnp.int32)],
                    jnp.full((L,), pb, jnp.int32))
                plsc.store_scatter(
                    mcnt, [jnp.full((L,), lp, jnp.int32)],
                    jnp.full((L,), cnt + 1, jnp.int32))
        return 0
    lax.fori_loop(0, MMAX // L, fill, 0)

    evs = [e8 * L + iota for e8 in range(EMBED // L)]

    def panel_cnt(q):
        return mcnt[pl.ds(q, L)][0]

    def drain_outs(q):
        cnt = panel_cnt(q)
        for ss in range(RING):
            @pl.when(q % jnp.int32(RING) == jnp.int32(ss))
            def _(ss=ss, cnt=cnt):
                for k in range(CAP):
                    @pl.when(k < cnt)
                    def _(ss=ss):
                        pltpu.make_async_copy(
                            out_hbm.at[pl.ds(0, EMBED)],
                            dbuf, semos[ss]).wait()

    def issue_panel(i):
        p = wid + jnp.int32(NW) * i
        valid = (p < jnp.int32(NPAN)) & (panel_cnt(i) > 0)

        @pl.when(valid)
        def _():
            t = p // jnp.int32(NPAN_T)
            j = p % jnp.int32(NPAN_T)

            @pl.when(j < jnp.int32(JTAIL))
            def _():
                for ss in range(RING):
                    @pl.when(i % jnp.int32(RING) == jnp.int32(ss))
                    def _(ss=ss):
                        for tt in range(NUM_NTYPE):
                            @pl.when(t == jnp.int32(tt))
                            def _(tt=tt, ss=ss):
                                pltpu.async_copy(
                                    embs[tt].at[:, pl.ds(j * PANW, PANW)],
                                    pbufs[ss], sems[ss])

    def extract_panel(q):
        cnt = panel_cnt(q)
        p = wid + jnp.int32(NW) * q
        j = p % jnp.int32(NPAN_T)
        normal = (p < jnp.int32(NPAN)) & (cnt > 0) & (j < jnp.int32(JTAIL))

        @pl.when(normal)
        def _():
            row0 = mslots[pl.ds(q * CAP, L)]
            row1 = mslots[pl.ds(q * CAP + L, L)]
            for ss in range(RING):
                @pl.when(q % jnp.int32(RING) == jnp.int32(ss))
                def _(ss=ss, row0=row0, row1=row1):
                    pltpu.make_async_copy(
                        e0.at[:, pl.ds(0, PANW)], pbufs[ss],
                        sems[ss]).wait()
                    for k in range(CAP):
                        @pl.when(k < cnt)
                        def _(k=k, ss=ss):
                            pb = row0[k] if k < L else row1[k - L]
                            b = pb >> 7
                            c = pb & jnp.int32(127)
                            cvec = jnp.full((L,), c, jnp.int32)
                            rbase = (ss * CAP + k) * EMBED
                            for e8 in range(EMBED // L):
                                rowbufs[pl.ds(rbase + e8 * L, L)] = (
                                    plsc.load_gather(pbufs[ss],
                                                     [evs[e8], cvec]))
                            pltpu.async_copy(
                                rowbufs.at[pl.ds(rbase, EMBED)],
                                out_hbm.at[pl.ds(b * EMBED, EMBED)],
                                semos[ss])

    def step(i, _):
        @pl.when(i >= jnp.int32(RING))
        def _():
            drain_outs(i - RING)

        @pl.when(i < jnp.int32(MYPAN))
        def _():
            issue_panel(i)

        @pl.when((i >= jnp.int32(LAG)) & (i < jnp.int32(MYPAN + LAG)))
        def _():
            extract_panel(i - LAG)
        return 0
    lax.fori_loop(0, MYPAN + LAG, step, 0)

    for o in range(RING):
        q = MYPAN + LAG - RING + o
        if q < MYPAN:
            drain_outs(jnp.int32(q))

    @pl.when(sm > 0)
    def _():
        for tt in range(NUM_NTYPE):
            pltpu.sync_copy(
                embs[tt].at[:, pl.ds(JTAIL * PANW, TBL - JTAIL * PANW)],
                ptail.at[pl.ds(tt * EMBED, EMBED)])

        def tail_elem(s, _):
            @pl.when(s < sm)
            def _():
                sel = plsc.load_gather(spc, [jnp.full((L,), s, jnp.int32)])
                spb = sel[0]
                b = spb >> 9
                t = (spb >> 7) & jnp.int32(3)
                c = spb & jnp.int32(127)
                cvec = jnp.full((L,), c, jnp.int32)
                for e8 in range(EMBED // L):
                    rowbufs[pl.ds(e8 * L, L)] = plsc.load_gather(
                        ptail, [t * jnp.int32(EMBED) + evs[e8], cvec])
                pltpu.async_copy(
                    rowbufs.at[pl.ds(0, EMBED)],
                    out_hbm.at[pl.ds(b * EMBED, EMBED)], semsp)
                pltpu.make_async_copy(
                    out_hbm.at[pl.ds(0, EMBED)],
                    dbuf, semsp).wait()
            return 0
        lax.fori_loop(0, SCAP, tail_elem, 0)


@jax.jit
def _run(node_tids, type_ids, emb0, emb1, emb2, emb3):
    mesh = plsc.VectorSubcoreMesh(
        core_axis_name="c", subcore_axis_name="s",
        num_cores=NC, num_subcores=NS)
    emb_t = [e.T for e in (emb0, emb1, emb2, emb3)]
    out1 = pl.kernel(
        _body,
        out_type=jax.ShapeDtypeStruct((B * EMBED,), jnp.float32),
        mesh=mesh,
        compiler_params=pltpu.CompilerParams(needs_layout_passes=False),
        scratch_types=[
            pltpu.VMEM((B,), jnp.int32),
            pltpu.VMEM((B,), jnp.int32),
            pltpu.VMEM((MMAX + 2 * L,), jnp.int32),
            pltpu.VMEM((MMAX + 2 * L,), jnp.int32),
            pltpu.VMEM((SCAP + 2 * L,), jnp.int32),
            pltpu.VMEM((MYPAN + 2 * L,), jnp.int32),
            pltpu.VMEM((MYPAN * CAP + 2 * L,), jnp.int32),
            [pltpu.VMEM((EMBED, PANW), jnp.float32)] * RING,
            pltpu.VMEM((RING * CAP * EMBED,), jnp.float32),
            pltpu.VMEM((EMBED,), jnp.float32),
            pltpu.VMEM((NUM_NTYPE * EMBED, L), jnp.float32),
            [pltpu.SemaphoreType.DMA] * RING,
            [pltpu.SemaphoreType.DMA] * RING,
            pltpu.SemaphoreType.DMA,
        ],
    )(node_tids, type_ids, *emb_t)
    return out1.reshape(B, EMBED)


def kernel(node_ids, node_tids, type_ids, emb0, emb1, emb2, emb3):
    del node_ids
    return _run(node_tids.astype(jnp.int32), type_ids.astype(jnp.int32),
                emb0, emb1, emb2, emb3)


# speedup vs baseline: 1.6381x; 1.6381x over previous
import jax
import jax.numpy as jnp
from jax import lax
from jax.experimental import pallas as pl
from jax.experimental.pallas import tpu as pltpu
from jax.experimental.pallas import tpu_sc as plsc

NUM_NTYPE = 4
TBL = 250000
EMBED = 64
B = 16384

NC, NS, L = 2, 16, 16
NW = NC * NS

PANW = 128
NPAN_T = (TBL + PANW - 1) // PANW
JTAIL = NPAN_T - 1
NPAN = NUM_NTYPE * NPAN_T
MYPAN = (NPAN + NW - 1) // NW
CAP = 20
MMAX = 1024
SCAP = 64
RING = 3
LAG = 2
IDC = 2048


def _body(tids_hbm, xids_hbm, e0, e1, e2, e3, out_hbm,
          tvb, xvb, cl_lp, cl_pb, spc, mcnt, mslots,
          pbufs, rowbufs, dbuf, tail16, sems, semo, semsp):
    embs = (e0, e1, e2, e3)
    wid = lax.axis_index("s") * NC + lax.axis_index("c")

    zeros = jnp.zeros((L,), jnp.int32)
    for z in range((MYPAN + 2 * L) // L):
        mcnt[pl.ds(z * L, L)] = zeros

    iota = lax.iota(jnp.int32, L)

    def chunk_scan(ch, offs):
        pltpu.sync_copy(tids_hbm.at[pl.ds(ch * (IDC // 128), IDC // 128)],
                        tvb)
        pltpu.sync_copy(xids_hbm.at[pl.ds(ch * (IDC // 128), IDC // 128)],
                        xvb)

        def scan(i, offs, ch=ch):
            off, soff = offs
            tvv = tvb[i >> 3, pl.ds((i & 7) * L, L)]
            xvv = xvb[i >> 3, pl.ds((i & 7) * L, L)]
            j = xvv >> 7
            p = tvv * jnp.int32(NPAN_T) + j
            mine = (p & jnp.int32(NW - 1)) == wid
            tail = j == jnp.int32(JTAIL)
            minen = mine & (~tail)
            mines = mine & tail
            bvec = ch * IDC + i * L + iota
            pb = (bvec << 7) | (xvv & jnp.int32(127))
            spb = (bvec << 9) | (tvv << 7) | (xvv & jnp.int32(127))

            incn = minen.astype(jnp.int32)
            cumn = plsc.cumsum(incn)
            destn = jnp.where(minen, off + cumn - incn,
                              jnp.int32(MMAX + L) + iota)
            plsc.store_scatter(cl_lp, [destn >> 7, destn & jnp.int32(127)],
                               p >> 5)
            plsc.store_scatter(cl_pb, [destn >> 7, destn & jnp.int32(127)],
                               pb)

            incs = mines.astype(jnp.int32)
            cums = plsc.cumsum(incs)
            dests = jnp.where(mines, soff + cums - incs,
                              jnp.int32(SCAP + L) + iota)
            plsc.store_scatter(spc, [dests], spb)
            return (off + cumn[L - 1], soff + cums[L - 1])
        return lax.fori_loop(0, IDC // L, scan, offs)
    m, sm = lax.fori_loop(0, B // IDC, chunk_scan,
                          (jnp.int32(0), jnp.int32(0)))

    def fill(g, _):
        lpv = cl_lp[g >> 3, pl.ds((g & 7) * L, L)]
        pbv = cl_pb[g >> 3, pl.ds((g & 7) * L, L)]
        for kk in range(L):
            idx = g * L + kk

            @pl.when(idx < m)
            def _(kk=kk, lpv=lpv, pbv=pbv):
                lp = lpv[kk]
                pb = pbv[kk]
                cnt = mcnt[pl.ds(lp, L)][0]
                plsc.store_scatter(
                    mslots, [jnp.full((L,), lp, jnp.int32),
                             jnp.full((L,), cnt, jnp.int32)],
                    jnp.full((L,), pb, jnp.int32))
                plsc.store_scatter(
                    mcnt, [jnp.full((L,), lp, jnp.int32)],
                    jnp.full((L,), cnt + 1, jnp.int32))
        return 0
    lax.fori_loop(0, MMAX // L, fill, 0)

    evs = [e8 * L + iota for e8 in range(EMBED // L)]

    def panel_cnt(q):
        return mcnt[pl.ds(q, L)][0]

    def issue_panel(i):
        p = wid + jnp.int32(NW) * i
        valid = (p < jnp.int32(NPAN)) & (panel_cnt(i) > 0)

        @pl.when(valid)
        def _():
            t = p // jnp.int32(NPAN_T)
            j = p % jnp.int32(NPAN_T)

            @pl.when(j < jnp.int32(JTAIL))
            def _():
                for ss in range(RING):
                    @pl.when(i % jnp.int32(RING) == jnp.int32(ss))
                    def _(ss=ss):
                        for tt in range(NUM_NTYPE):
                            @pl.when(t == jnp.int32(tt))
                            def _(tt=tt, ss=ss):
                                pltpu.async_copy(
                                    embs[tt].at[:, pl.ds(j * PANW, PANW)],
                                    pbufs[ss], sems[ss])

    def extract_panel(q, hb):
        cnt = panel_cnt(q)
        p = wid + jnp.int32(NW) * q
        j = p % jnp.int32(NPAN_T)
        normal = (p < jnp.int32(NPAN)) & (cnt > 0) & (j < jnp.int32(JTAIL))

        @pl.when(normal)
        def _():
            row0 = mslots[q, pl.ds(0, L)]
            row1 = mslots[q, pl.ds(L, L)]
            for ss in range(RING):
                @pl.when(q % jnp.int32(RING) == jnp.int32(ss))
                def _(ss=ss, row0=row0, row1=row1):
                    pltpu.make_async_copy(
                        e0.at[:, pl.ds(0, PANW)], pbufs[ss],
                        sems[ss]).wait()
                    for k in range(CAP):
                        @pl.when(k < cnt)
                        def _(k=k, ss=ss):
                            pb = row0[k] if k < L else row1[k - L]
                            b = pb >> 7
                            c = pb & jnp.int32(127)
                            cvec = jnp.full((L,), c, jnp.int32)
                            hh = hb + jnp.int32(k)
                            hrow = hh >> 1
                            hcol = (hh & jnp.int32(1)) * jnp.int32(EMBED)
                            for e8 in range(EMBED // L):
                                rowbufs[hrow, pl.ds(hcol + e8 * L, L)] = (
                                    plsc.load_gather(pbufs[ss],
                                                     [evs[e8], cvec]))
                            pltpu.async_copy(
                                rowbufs.at[hrow, pl.ds(hcol, EMBED)],
                                out_hbm.at[pl.ds(b * EMBED, EMBED)],
                                semo)
        return hb + jnp.where(normal, cnt, jnp.int32(0))

    def step(i, hb):
        @pl.when(i < jnp.int32(MYPAN))
        def _():
            issue_panel(i)

        return lax.cond((i >= jnp.int32(LAG)) & (i < jnp.int32(MYPAN + LAG)),
                        lambda: extract_panel(i - LAG, hb),
                        lambda: hb)
    lax.fori_loop(0, MYPAN + LAG, step, jnp.int32(0))

    def drain(k, _):
        @pl.when(k < m)
        def _():
            pltpu.make_async_copy(
                out_hbm.at[pl.ds(0, EMBED)], dbuf, semo).wait()
        return 0
    lax.fori_loop(0, MMAX, drain, 0)

    @pl.when(sm > 0)
    def _():
        def tail_elem(s, _):
            @pl.when(s < sm)
            def _():
                sel = plsc.load_gather(spc, [jnp.full((L,), s, jnp.int32)])
                spb = sel[0]
                b = spb >> 9
                t = (spb >> 7) & jnp.int32(3)
                c = spb & jnp.int32(127)
                cvec = jnp.full((L,), c, jnp.int32)
                for tt in range(NUM_NTYPE):
                    @pl.when(t == jnp.int32(tt))
                    def _(tt=tt):
                        pltpu.sync_copy(
                            embs[tt].at[:, pl.ds(JTAIL * PANW,
                                                 TBL - JTAIL * PANW)],
                            tail16)
                for e8 in range(EMBED // L):
                    rowbufs[jnp.int32(0), pl.ds(e8 * L, L)] = (
                        plsc.load_gather(tail16, [evs[e8], cvec]))
                pltpu.async_copy(
                    rowbufs.at[jnp.int32(0), pl.ds(0, EMBED)],
                    out_hbm.at[pl.ds(b * EMBED, EMBED)], semsp)
                pltpu.make_async_copy(
                    out_hbm.at[pl.ds(0, EMBED)], dbuf, semsp).wait()
            return 0
        lax.fori_loop(0, SCAP, tail_elem, 0)


@jax.jit
def _run(node_tids, type_ids, emb0, emb1, emb2, emb3):
    mesh = plsc.VectorSubcoreMesh(
        core_axis_name="c", subcore_axis_name="s",
        num_cores=NC, num_subcores=NS)
    emb_t = [e.T for e in (emb0, emb1, emb2, emb3)]
    node_tids = node_tids.reshape(B // 128, 128)
    type_ids = type_ids.reshape(B // 128, 128)
    out1 = pl.kernel(
        _body,
        out_type=jax.ShapeDtypeStruct((B * EMBED,), jnp.float32),
        mesh=mesh,
        compiler_params=pltpu.CompilerParams(needs_layout_passes=False),
        scratch_types=[
            pltpu.VMEM((IDC // 128, 128), jnp.int32),
            pltpu.VMEM((IDC // 128, 128), jnp.int32),
            pltpu.VMEM((MMAX // 128 + 1, 128), jnp.int32),
            pltpu.VMEM((MMAX // 128 + 1, 128), jnp.int32),
            pltpu.VMEM((SCAP + 2 * L,), jnp.int32),
            pltpu.VMEM((MYPAN + 2 * L,), jnp.int32),
            pltpu.VMEM((MYPAN, 128), jnp.int32),
            [pltpu.VMEM((EMBED, PANW), jnp.float32)] * RING,
            pltpu.VMEM((MMAX // 2 - 128, 128), jnp.float32),
            pltpu.VMEM((EMBED,), jnp.float32),
            pltpu.VMEM((EMBED, L), jnp.float32),
            [pltpu.SemaphoreType.DMA] * RING,
            pltpu.SemaphoreType.DMA,
            pltpu.SemaphoreType.DMA,
        ],
    )(node_tids, type_ids, *emb_t)
    return out1.reshape(B, EMBED)


def kernel(node_ids, node_tids, type_ids, emb0, emb1, emb2, emb3):
    del node_ids
    return _run(node_tids.astype(jnp.int32), type_ids.astype(jnp.int32),
                emb0, emb1, emb2, emb3)
